# Initial kernel scaffold; baseline (speedup 1.0000x reference)
import jax
import jax.numpy as jnp
from jax.experimental import pallas as pl


def kernel(updates, mask):
    raise NotImplementedError("write your pallas kernel here")


# speedup vs baseline: 15.8112x; 15.8112x over previous
import functools

import jax
import jax.numpy as jnp
from jax import lax
from jax.experimental import pallas as pl
from jax.experimental.pallas import tpu as pltpu
from jax.experimental.pallas import tpu_sc as plsc

B, H, W, C = 4, 112, 112, 192
oH, oW = 2 * H, 2 * W
HW = H * W
oHW = oH * oW
CB = 16
NCB = C // CB
NC, NS = 2, 16
NTASK = B * NCB
TPC = NTASK // NC
RPT = HW // NS
ORPT = oHW // NS
NCHUNK = RPT * CB // 128
ZCH = 6272
OCH = 8
ORC = ORPT // OCH

_mesh = plsc.VectorSubcoreMesh(core_axis_name="c", subcore_axis_name="s")


@functools.partial(
    pl.kernel,
    mesh=_mesh,
    out_type=jax.ShapeDtypeStruct((B, oHW, C), jnp.float32),
    compiler_params=pltpu.CompilerParams(use_tc_tiling_on_sc=False),
    scratch_types=[
        pltpu.VMEM((RPT, CB), jnp.float32),
        pltpu.VMEM((RPT, CB), jnp.int32),
        pltpu.VMEM((NCHUNK, 128), jnp.float32),
        pltpu.VMEM((NCHUNK, 128), jnp.int32),
        pltpu.VMEM((ORC * CB,), jnp.float32),
        pltpu.VMEM((ORC, CB), jnp.float32),
        pltpu.VMEM((ZCH,), jnp.float32),
        pltpu.VMEM_SHARED((oHW * CB,), jnp.float32),
    ],
)
def _unpool_sc(upd_hbm, mask_hbm, out_hbm, u_raw, m_raw, uv, iv, st1, st2,
               zbuf, acc):
    core = lax.axis_index("c")
    sid = lax.axis_index("s")

    zero16 = jnp.zeros((16,), jnp.float32)

    def zinit(i, carry):
        zbuf[pl.ds(i * 16, 16)] = zero16
        return carry

    lax.fori_loop(0, ZCH // 16, zinit, 0)

    lanes = lax.iota(jnp.int32, 16)
    third = jnp.float32(1.0 / 3.0)

    def task_body(t, carry):
        task = core * TPC + t
        b = task // NCB
        cb0 = (task % NCB) * CB
        r0 = sid * RPT
        o0 = sid * ORPT

        for z in range(ORPT * CB // ZCH):
            pltpu.sync_copy(zbuf, acc.at[pl.ds(sid * (ORPT * CB) + z * ZCH, ZCH)])
        plsc.subcore_barrier()

        pltpu.sync_copy(upd_hbm.at[b, pl.ds(r0, RPT), pl.ds(cb0, CB)], u_raw)
        pltpu.sync_copy(mask_hbm.at[b, pl.ds(r0, RPT), pl.ds(cb0, CB)], m_raw)

        def crow(j, carry):
            m = m_raw[j, :]
            u = u_raw[j, :]
            t6 = lax.shift_right_logical(m, 6)
            s = (t6.astype(jnp.float32) * third).astype(jnp.int32)
            idx = s * CB + lanes
            cj = j // 8
            off = (j % 8) * 16
            iv[cj, pl.ds(off, 16)] = idx
            uv[cj, pl.ds(off, 16)] = u
            return carry

        lax.fori_loop(0, RPT, crow, 0)

        def cscat(cj, carry):
            pltpu.sync_copy(uv.at[cj], acc.at[iv.at[cj]], add=True)
            return carry

        lax.fori_loop(0, NCHUNK, cscat, 0)
        plsc.subcore_barrier()

        def cout(r, carry):
            pltpu.sync_copy(acc.at[pl.ds((o0 + r * ORC) * CB, ORC * CB)], st1)

            def regroup(k, c2):
                st2[k, :] = st1[pl.ds(k * CB, CB)]
                return c2

            lax.fori_loop(0, ORC, regroup, 0)
            pltpu.sync_copy(
                st2, out_hbm.at[b, pl.ds(o0 + r * ORC, ORC), pl.ds(cb0, CB)])
            return carry

        lax.fori_loop(0, OCH, cout, 0)
        return carry

    lax.fori_loop(0, TPC, task_body, 0)


def kernel(updates, mask):
    u = updates.reshape(B, HW, C)
    m = mask.astype(jnp.int32).reshape(B, HW, C)
    out = _unpool_sc(u, m)
    return out.reshape(B, oH, oW, C)
